# baseline (speedup 1.0000x reference)
import jax
import jax.numpy as jnp
from jax import lax
from jax.experimental import pallas as pl
from jax.experimental.pallas import tpu as pltpu
from jax.experimental.pallas import tpu_sc as plsc

N = 100000
D = 128
NW = 32
C = 128
NCH = 25
RPW = NCH * C
NPAD = NW * RPW
NBUF = 2
LAST_FULL = (N - (NW - 1) * RPW) // C
TAIL = N - (NW - 1) * RPW - LAST_FULL * C


def _gather_body(x_hbm, idx_hbm, out_hbm, idx_v, r0, r1, g0, g1, s0, s1):
    wid = lax.axis_index("s") * 2 + lax.axis_index("c")
    base = pl.multiple_of(wid * RPW, RPW)
    pltpu.sync_copy(idx_hbm.at[pl.ds(base, RPW)], idx_v)

    bufs = (r0, r1)
    gsems = (g0, g1)
    ssems = (s0, s1)

    def gather_desc(k):
        b = k % NBUF
        return pltpu.make_async_copy(
            x_hbm.at[idx_v.at[pl.ds(k * C, C)]], bufs[b], gsems[b])

    def full_store_desc(k):
        b = k % NBUF
        return pltpu.make_async_copy(
            bufs[b], out_hbm.at[pl.ds(base + k * C, C)], ssems[b])

    def tail_store_desc(k):
        b = k % NBUF
        return pltpu.make_async_copy(
            bufs[b].at[pl.ds(0, TAIL)],
            out_hbm.at[pl.ds(base + k * C, TAIL)], ssems[b])

    def store_op(k, op):
        if k < LAST_FULL:
            getattr(full_store_desc(k), op)()
        elif k == LAST_FULL:
            @pl.when(wid != NW - 1)
            def _():
                getattr(full_store_desc(k), op)()

            @pl.when(wid == NW - 1)
            def _():
                getattr(tail_store_desc(k), op)()
        else:
            @pl.when(wid != NW - 1)
            def _():
                getattr(full_store_desc(k), op)()

    gather_desc(0).start()
    for k in range(NCH):
        if k + 1 < NCH:
            if k + 1 >= NBUF:
                store_op(k + 1 - NBUF, "wait")
            gather_desc(k + 1).start()
        gather_desc(k).wait()
        store_op(k, "start")
    for j in range(max(0, NCH - NBUF), NCH):
        store_op(j, "wait")


@jax.jit
def _gather(x, idx):
    mesh = plsc.VectorSubcoreMesh(core_axis_name="c", subcore_axis_name="s")
    f = pl.kernel(
        _gather_body,
        out_type=jax.ShapeDtypeStruct((N, D), jnp.float32),
        mesh=mesh,
        scratch_types=[
            pltpu.VMEM((RPW,), jnp.int32),
            pltpu.VMEM((C, D), jnp.float32),
            pltpu.VMEM((C, D), jnp.float32),
            pltpu.SemaphoreType.DMA,
            pltpu.SemaphoreType.DMA,
            pltpu.SemaphoreType.DMA,
            pltpu.SemaphoreType.DMA,
        ],
    )
    return f(x, idx)


def kernel(x, cell_type_indices, permutations):
    idx = permutations.reshape(-1).astype(jnp.int32)
    idx = jnp.concatenate([idx, jnp.zeros((NPAD - N,), jnp.int32)])
    return _gather(x, idx)


# speedup vs baseline: 1.0167x; 1.0167x over previous
import jax
import jax.numpy as jnp
from jax import lax
from jax.experimental import pallas as pl
from jax.experimental.pallas import tpu as pltpu
from jax.experimental.pallas import tpu_sc as plsc

N = 100000
D = 128
NW = 32
C = 128
NCH = 25
RPW = NCH * C
NPAD = NW * RPW
NBUF = 5
NGRP = NCH // NBUF
LAST_FULL = (N - (NW - 1) * RPW) // C
TAIL = N - (NW - 1) * RPW - LAST_FULL * C


def _gather_body(x_hbm, idx_hbm, out_hbm, idx_v, bufs, gsems, ssems):
    wid = lax.axis_index("s") * 2 + lax.axis_index("c")
    base = pl.multiple_of(wid * RPW, RPW)
    last = wid == NW - 1
    pltpu.sync_copy(idx_hbm.at[pl.ds(base, RPW)], idx_v)

    def gather_desc(k, b):
        off = pl.multiple_of(k * C, C)
        return pltpu.make_async_copy(
            x_hbm.at[idx_v.at[pl.ds(off, C)]], bufs[b], gsems[b])

    def store_op(k, b, op):
        @pl.when(jnp.logical_not(last) | (k < LAST_FULL))
        def _():
            getattr(pltpu.make_async_copy(
                bufs[b], out_hbm.at[pl.ds(base + k * C, C)], ssems[b]), op)()

        @pl.when(last & (k == LAST_FULL))
        def _():
            getattr(pltpu.make_async_copy(
                bufs[b].at[pl.ds(0, TAIL)],
                out_hbm.at[pl.ds(base + k * C, TAIL)], ssems[b]), op)()

    for b in range(NBUF):
        gather_desc(b, b).start()

    def group(g, carry):
        for b in range(NBUF):
            k = g * NBUF + b
            gather_desc(k, b).wait()
            store_op(k, b, "start")
        for b in range(NBUF):
            k = g * NBUF + b

            @pl.when(g < NGRP - 1)
            def _():
                store_op(k, b, "wait")
                gather_desc(k + NBUF, b).start()
        return carry

    lax.fori_loop(0, NGRP, group, 0)

    for b in range(NBUF):
        store_op((NGRP - 1) * NBUF + b, b, "wait")


@jax.jit
def _gather(x, idx):
    mesh = plsc.VectorSubcoreMesh(core_axis_name="c", subcore_axis_name="s")

    def body(x_hbm, idx_hbm, out_hbm, idx_v, *rest):
        bufs = rest[0:NBUF]
        gsems = rest[NBUF:2 * NBUF]
        ssems = rest[2 * NBUF:3 * NBUF]
        _gather_body(x_hbm, idx_hbm, out_hbm, idx_v, bufs, gsems, ssems)

    f = pl.kernel(
        body,
        out_type=jax.ShapeDtypeStruct((N, D), jnp.float32),
        mesh=mesh,
        scratch_types=(
            [pltpu.VMEM((RPW,), jnp.int32)]
            + [pltpu.VMEM((C, D), jnp.float32)] * NBUF
            + [pltpu.SemaphoreType.DMA] * (2 * NBUF)
        ),
    )
    return f(x, idx)


def kernel(x, cell_type_indices, permutations):
    idx = permutations.reshape(-1).astype(jnp.int32)
    idx = jnp.concatenate([idx, jnp.zeros((NPAD - N,), jnp.int32)])
    return _gather(x, idx)


# speedup vs baseline: 1.9320x; 1.9003x over previous
import jax
import jax.numpy as jnp
from jax import lax
from jax.experimental import pallas as pl
from jax.experimental.pallas import tpu as pltpu
from jax.experimental.pallas import tpu_sc as plsc

N = 100000
D = 128
NW = 32
C = 128
NCH = 25
RPW = NCH * C
NPAD = NW * RPW
LAST_FULL = (N - (NW - 1) * RPW) // C
TAIL = N - (NW - 1) * RPW - LAST_FULL * C


def _gather_body(x_hbm, idx_hbm, out_hbm, idx_v, rows_v, sem):
    wid = lax.axis_index("s") * 2 + lax.axis_index("c")
    base = pl.multiple_of(wid * RPW, RPW)
    pltpu.sync_copy(idx_hbm.at[pl.ds(base, RPW)], idx_v)
    nfull = jnp.where(wid == NW - 1, LAST_FULL, NCH)

    def chunk(k, carry):
        off = pl.multiple_of(k * C, C)
        pltpu.async_copy(x_hbm.at[idx_v.at[pl.ds(off, C)]], rows_v, sem).wait()
        pltpu.sync_copy(rows_v, out_hbm.at[pl.ds(base + off, C)])
        return carry

    lax.fori_loop(0, nfull, chunk, 0)

    @pl.when(wid == NW - 1)
    def _():
        off = LAST_FULL * C
        pltpu.async_copy(x_hbm.at[idx_v.at[pl.ds(off, C)]], rows_v, sem).wait()
        pltpu.sync_copy(
            rows_v.at[pl.ds(0, TAIL)],
            out_hbm.at[pl.ds(base + off, TAIL)],
        )


@jax.jit
def _gather(x, idx):
    mesh = plsc.VectorSubcoreMesh(core_axis_name="c", subcore_axis_name="s")
    f = pl.kernel(
        _gather_body,
        out_type=jax.ShapeDtypeStruct((N, D), jnp.float32),
        mesh=mesh,
        scratch_types=[
            pltpu.VMEM((RPW,), jnp.int32),
            pltpu.VMEM((C, D), jnp.float32),
            pltpu.SemaphoreType.DMA,
        ],
    )
    return f(x, idx)


def kernel(x, cell_type_indices, permutations):
    idx = permutations.reshape(-1).astype(jnp.int32)
    idx = jnp.concatenate([idx, jnp.zeros((NPAD - N,), jnp.int32)])
    return _gather(x, idx)


# speedup vs baseline: 2.3230x; 1.2024x over previous
import jax
import jax.numpy as jnp
from jax import lax
from jax.experimental import pallas as pl
from jax.experimental.pallas import tpu as pltpu
from jax.experimental.pallas import tpu_sc as plsc

N = 100000
D = 128
NW = 32
C = 128
NCH = 25
RPW = NCH * C
NPAD = NW * RPW
LAST_FULL = (N - (NW - 1) * RPW) // C
TAIL = N - (NW - 1) * RPW - LAST_FULL * C
NPAIR = (NCH - 1) // 2
NPAIR_LAST = LAST_FULL // 2


def _gather_body(x_hbm, idx_hbm, out_hbm, idx_v, buf0, buf1, g0, g1):
    wid = lax.axis_index("s") * 2 + lax.axis_index("c")
    base = pl.multiple_of(wid * RPW, RPW)
    last = wid == NW - 1
    pltpu.sync_copy(idx_hbm.at[pl.ds(base, RPW)], idx_v)

    def gather(k, buf, sem):
        off = pl.multiple_of(k * C, C)
        return pltpu.async_copy(x_hbm.at[idx_v.at[pl.ds(off, C)]], buf, sem)

    def gwait(k, buf, sem):
        off = pl.multiple_of(k * C, C)
        pltpu.make_async_copy(x_hbm.at[idx_v.at[pl.ds(off, C)]], buf, sem).wait()

    def store(k, buf):
        pltpu.sync_copy(buf, out_hbm.at[pl.ds(base + k * C, C)])

    npair = jnp.where(last, NPAIR_LAST, NPAIR)
    gather(0, buf0, g0)

    def pair(i, carry):
        k0 = 2 * i
        gather(k0 + 1, buf1, g1)
        gwait(k0, buf0, g0)
        store(k0, buf0)
        gather(k0 + 2, buf0, g0)
        gwait(k0 + 1, buf1, g1)
        store(k0 + 1, buf1)
        return carry

    lax.fori_loop(0, npair, pair, 0)

    @pl.when(jnp.logical_not(last))
    def _():
        gwait(2 * NPAIR, buf0, g0)
        store(2 * NPAIR, buf0)

    @pl.when(last)
    def _():
        gwait(LAST_FULL, buf0, g0)
        pltpu.sync_copy(
            buf0.at[pl.ds(0, TAIL)],
            out_hbm.at[pl.ds(base + LAST_FULL * C, TAIL)],
        )


@jax.jit
def _gather(x, idx):
    mesh = plsc.VectorSubcoreMesh(core_axis_name="c", subcore_axis_name="s")
    f = pl.kernel(
        _gather_body,
        out_type=jax.ShapeDtypeStruct((N, D), jnp.float32),
        mesh=mesh,
        scratch_types=[
            pltpu.VMEM((RPW,), jnp.int32),
            pltpu.VMEM((C, D), jnp.float32),
            pltpu.VMEM((C, D), jnp.float32),
            pltpu.SemaphoreType.DMA,
            pltpu.SemaphoreType.DMA,
        ],
    )
    return f(x, idx)


def kernel(x, cell_type_indices, permutations):
    idx = permutations.reshape(-1).astype(jnp.int32)
    idx = jnp.concatenate([idx, jnp.zeros((NPAD - N,), jnp.int32)])
    return _gather(x, idx)


# speedup vs baseline: 2.5018x; 1.0770x over previous
import jax
import jax.numpy as jnp
from jax import lax
from jax.experimental import pallas as pl
from jax.experimental.pallas import tpu as pltpu
from jax.experimental.pallas import tpu_sc as plsc

N = 100000
D = 128
NW = 32
C = 128
NCH = 25
RPW = NCH * C
NPAD = NW * RPW
NBUF = 4
LAST_FULL = (N - (NW - 1) * RPW) // C
TAIL = N - (NW - 1) * RPW - LAST_FULL * C


def _gather_body(x_hbm, idx_hbm, out_hbm, idx_v, *rest):
    bufs = rest[0:NBUF]
    gsems = rest[NBUF:2 * NBUF]
    ssems = rest[2 * NBUF:3 * NBUF]

    wid = lax.axis_index("s") * 2 + lax.axis_index("c")
    base = pl.multiple_of(wid * RPW, RPW)
    last = wid == NW - 1
    pltpu.sync_copy(idx_hbm.at[pl.ds(base, RPW)], idx_v)

    def g_desc(k, b):
        off = pl.multiple_of(k * C, C)
        return pltpu.make_async_copy(
            x_hbm.at[idx_v.at[pl.ds(off, C)]], bufs[b], gsems[b])

    def s_desc(k, b):
        return pltpu.make_async_copy(
            bufs[b], out_hbm.at[pl.ds(base + k * C, C)], ssems[b])

    def sg(k, b):
        g_desc(k, b).start()

    def wg(k, b):
        g_desc(k, b).wait()

    def ss(k, b):
        s_desc(k, b).start()

    def ws(k, b):
        s_desc(k, b).wait()

    @pl.when(jnp.logical_not(last))
    def _():
        sg(0, 0)
        sg(1, 1)
        sg(2, 2)
        wg(0, 0)
        ss(0, 0)
        sg(3, 3)
        wg(1, 1)
        ss(1, 1)
        ws(0, 0)
        sg(4, 0)
        wg(2, 2)
        ss(2, 2)
        ws(1, 1)
        sg(5, 1)
        wg(3, 3)
        ss(3, 3)

        def quad(i, carry):
            for j in range(4):
                k = 4 * i + j
                ws(k - 2, (j + 2) % 4)
                sg(k + 2, (j + 2) % 4)
                wg(k, j)
                ss(k, j)
            return carry

        lax.fori_loop(1, 5, quad, 0)

        ws(18, 2)
        sg(22, 2)
        wg(20, 0)
        ss(20, 0)
        ws(19, 3)
        sg(23, 3)
        wg(21, 1)
        ss(21, 1)
        ws(20, 0)
        sg(24, 0)
        wg(22, 2)
        ss(22, 2)
        ws(21, 1)
        wg(23, 3)
        ss(23, 3)
        ws(22, 2)
        wg(24, 0)
        ss(24, 0)
        ws(23, 3)
        ws(24, 0)

    @pl.when(last)
    def _():
        def chunk(k, carry):
            off = pl.multiple_of(k * C, C)
            pltpu.async_copy(
                x_hbm.at[idx_v.at[pl.ds(off, C)]], bufs[0], gsems[0]).wait()
            pltpu.sync_copy(bufs[0], out_hbm.at[pl.ds(base + off, C)])
            return carry

        lax.fori_loop(0, LAST_FULL, chunk, 0)
        off = LAST_FULL * C
        pltpu.async_copy(
            x_hbm.at[idx_v.at[pl.ds(off, C)]], bufs[0], gsems[0]).wait()
        pltpu.sync_copy(
            bufs[0].at[pl.ds(0, TAIL)],
            out_hbm.at[pl.ds(base + off, TAIL)],
        )


@jax.jit
def _gather(x, idx):
    mesh = plsc.VectorSubcoreMesh(core_axis_name="c", subcore_axis_name="s")
    f = pl.kernel(
        _gather_body,
        out_type=jax.ShapeDtypeStruct((N, D), jnp.float32),
        mesh=mesh,
        scratch_types=(
            [pltpu.VMEM((RPW,), jnp.int32)]
            + [pltpu.VMEM((C, D), jnp.float32)] * NBUF
            + [pltpu.SemaphoreType.DMA] * (2 * NBUF)
        ),
    )
    return f(x, idx)


def kernel(x, cell_type_indices, permutations):
    idx = permutations.reshape(-1).astype(jnp.int32)
    idx = jnp.concatenate([idx, jnp.zeros((NPAD - N,), jnp.int32)])
    return _gather(x, idx)
